# baseline (speedup 1.0000x reference)
import functools

import jax
import jax.numpy as jnp
from jax import lax
from jax.experimental import pallas as pl
from jax.experimental.pallas import tpu as pltpu
from jax.experimental.pallas import tpu_sc as plsc

_B, _T, _C = 128, 50, 8192
_NC, _NS = 1, 16
_NW = _NC * _NS
_BPW = _B // _NW
_W = 512
_NST = _C // _W
_L = 16


def _sc_onehot_body(lab_hbm, prob_hbm, out_hbm, buf, labv, probv, sems):
    wid = lax.axis_index("s") * _NC + lax.axis_index("c")
    b0 = wid * _BPW

    lane = lax.iota(jnp.int32, _L)
    zvec = jnp.zeros((_L,), jnp.float32)

    pltpu.sync_copy(lab_hbm.at[pl.ds(b0, _BPW), :], labv)
    pltpu.sync_copy(prob_hbm, probv)
    pvec = probv[...]

    def chunk_dma(b_loc, st, s):
        return pltpu.make_async_copy(
            buf.at[s],
            out_hbm.at[b0 + b_loc, :, pl.ds(st * _W, _W)],
            sems.at[s])

    def chunk_body(chunk, _):
        b_loc = chunk // _NST
        st = lax.rem(chunk, _NST)
        s = lax.rem(chunk, 2)
        c0 = st * _W

        @pl.when(chunk >= 2)
        def _():
            chunk_dma((chunk - 2) // _NST, lax.rem(chunk - 2, _NST), s).wait()

        def row(t, _):
            rowref = buf.at[s, t]
            tgt = labv[b_loc, pl.ds(t * _L, _L)] - lane - c0
            for w in range(_W // _L):
                rowref[pl.ds(w * _L, _L)] = jnp.where(tgt == 0, pvec, zvec)
                tgt = tgt - _L
            return 0
        lax.fori_loop(0, _T, row, 0)
        chunk_dma(b_loc, st, s).start()
        return 0

    nch = _BPW * _NST
    lax.fori_loop(0, nch, chunk_body, 0)
    for chunk in (nch - 2, nch - 1):
        chunk_dma(chunk // _NST, chunk % _NST, chunk % 2).wait()


_sc_onehot = functools.partial(
    pl.kernel,
    out_type=jax.ShapeDtypeStruct((_B, _T, _C), jnp.float32),
    mesh=plsc.VectorSubcoreMesh(
        core_axis_name="c", subcore_axis_name="s",
        num_cores=_NC, num_subcores=_NS),
    scratch_types=[
        pltpu.VMEM((2, _T, _W), jnp.float32),
        pltpu.VMEM((_BPW, _T * _L), jnp.int32),
        pltpu.VMEM((_L,), jnp.float32),
        pltpu.SemaphoreType.DMA((2,)),
    ],
    compiler_params=pltpu.CompilerParams(use_tc_tiling_on_sc=True),
)(_sc_onehot_body)


def kernel(LLM_label, prob):
    blab = jnp.broadcast_to(
        LLM_label.astype(jnp.int32)[..., None], (_B, _T, _L))
    prob16 = jnp.broadcast_to(prob.astype(jnp.float32), (_L,))
    return _sc_onehot(blab.reshape(_B, _T * _L), prob16)


# speedup vs baseline: 1.3051x; 1.3051x over previous
import functools

import jax
import jax.numpy as jnp
from jax import lax
from jax.experimental import pallas as pl
from jax.experimental.pallas import tpu as pltpu
from jax.experimental.pallas import tpu_sc as plsc

_B, _T, _C = 128, 50, 8192
_NC, _NS = 2, 16
_NW = _NC * _NS
_BPW = _B // _NW
_W = 512
_NST = _C // _W
_L = 16


def _sc_onehot_body(lab_hbm, prob_hbm, out_hbm, buf, labv, probv, sems):
    wid = lax.axis_index("s") * _NC + lax.axis_index("c")
    b0 = wid * _BPW

    lane = lax.iota(jnp.int32, _L)
    zvec = jnp.zeros((_L,), jnp.float32)

    pltpu.sync_copy(lab_hbm.at[pl.ds(b0, _BPW), :], labv)
    pltpu.sync_copy(prob_hbm, probv)
    pvec = probv[...]

    def chunk_dma(b_loc, st, s):
        return pltpu.make_async_copy(
            buf.at[s],
            out_hbm.at[b0 + b_loc, :, pl.ds(st * _W, _W)],
            sems.at[s])

    def chunk_body(chunk, _):
        b_loc = chunk // _NST
        st = lax.rem(chunk, _NST)
        s = lax.rem(chunk, 2)
        c0 = st * _W

        @pl.when(chunk >= 2)
        def _():
            chunk_dma((chunk - 2) // _NST, lax.rem(chunk - 2, _NST), s).wait()

        def row(t, _):
            rowref = buf.at[s, t]
            tgt = labv[b_loc, pl.ds(t * _L, _L)] - lane - c0
            for w in range(_W // _L):
                rowref[pl.ds(w * _L, _L)] = jnp.where(tgt == 0, pvec, zvec)
                tgt = tgt - _L
            return 0
        lax.fori_loop(0, _T, row, 0)
        chunk_dma(b_loc, st, s).start()
        return 0

    nch = _BPW * _NST
    lax.fori_loop(0, nch, chunk_body, 0)
    for chunk in (nch - 2, nch - 1):
        chunk_dma(chunk // _NST, chunk % _NST, chunk % 2).wait()


_sc_onehot = functools.partial(
    pl.kernel,
    out_type=jax.ShapeDtypeStruct((_B, _T, _C), jnp.float32),
    mesh=plsc.VectorSubcoreMesh(
        core_axis_name="c", subcore_axis_name="s",
        num_cores=_NC, num_subcores=_NS),
    scratch_types=[
        pltpu.VMEM((2, _T, _W), jnp.float32),
        pltpu.VMEM((_BPW, _T * _L), jnp.int32),
        pltpu.VMEM((_L,), jnp.float32),
        pltpu.SemaphoreType.DMA((2,)),
    ],
    compiler_params=pltpu.CompilerParams(use_tc_tiling_on_sc=True),
)(_sc_onehot_body)


def kernel(LLM_label, prob):
    blab = jnp.broadcast_to(
        LLM_label.astype(jnp.int32)[..., None], (_B, _T, _L))
    prob16 = jnp.broadcast_to(prob.astype(jnp.float32), (_L,))
    return _sc_onehot(blab.reshape(_B, _T * _L), prob16)
